# baseline (speedup 1.0000x reference)
import functools

import jax
import jax.numpy as jnp
from jax import lax
from jax.experimental import pallas as pl
from jax.experimental.pallas import tpu as pltpu
from jax.experimental.pallas import tpu_sc as plsc

RING_ID_START = 4
RING_ID_END = 104
NSLOT = 128
B, S, E, H = 16, 2048, 256, 64
ROUT = RING_ID_END - RING_ID_START

_NC, _NS = 2, 16
_NW = _NC * _NS
_BPW = (B * NSLOT) // _NW


def _winner_body(seq_ref, flat_ref, mask_ref):
    for b in range(B):
        row = seq_ref[b, :]
        valid = (row >= RING_ID_START) & (row <= RING_ID_END - 1)
        slot = jnp.where(valid, row - RING_ID_START, -1)
        slotb = jnp.broadcast_to(slot[None, :], (NSLOT, S))
        jcol = lax.broadcasted_iota(jnp.int32, (NSLOT, S), 0)
        siota = lax.broadcasted_iota(jnp.int32, (NSLOT, S), 1)
        vals = jnp.where(slotb == jcol, siota, -1)
        winner = jnp.max(vals, axis=1)
        flat_ref[b, :] = jnp.maximum(winner, 0) + b * S
        mask_ref[b, 0, :] = (winner >= 0).astype(jnp.float32)


def _winner_call(sequences):
    return pl.pallas_call(
        _winner_body,
        out_shape=(
            jax.ShapeDtypeStruct((B, NSLOT), jnp.int32),
            jax.ShapeDtypeStruct((B, 1, NSLOT), jnp.float32),
        ),
    )(sequences)


def _gather_sc_body(table_hbm, idx_hbm, out_hbm, idx_v, rows_v, sem):
    wid = lax.axis_index("s") * _NC + lax.axis_index("c")
    base = wid * _BPW
    pltpu.sync_copy(idx_hbm.at[pl.ds(base, _BPW)], idx_v)
    pltpu.async_copy(table_hbm.at[idx_v], rows_v, sem).wait()
    pltpu.sync_copy(rows_v, out_hbm.at[pl.ds(base, _BPW)])


@functools.cache
def _gather_sc_kernel():
    return pl.kernel(
        _gather_sc_body,
        mesh=plsc.VectorSubcoreMesh(core_axis_name="c", subcore_axis_name="s"),
        out_type=jax.ShapeDtypeStruct((B * NSLOT, E), jnp.float32),
        scratch_types=[
            pltpu.VMEM((_BPW,), jnp.int32),
            pltpu.VMEM((_BPW, E), jnp.float32),
            pltpu.SemaphoreType.DMA,
        ],
    )


def _gather_sc(table, idx):
    return _gather_sc_kernel()(table, idx)


def _logits_body(x_ref, xg_ref, m_ref, w0t_ref, b0_ref, w1t_ref, b1_ref,
                 out_ref):
    scale = H ** -0.5
    xg = xg_ref[0]
    m = m_ref[0]
    out1t = lax.dot_general(xg, w1t_ref[...], (((1,), (1,)), ((), ())),
                            preferred_element_type=jnp.float32,
                            precision=lax.Precision.HIGHEST) + b1_ref[...]
    n = lax.dot_general(w0t_ref[...], out1t, (((0,), (1,)), ((), ())),
                        preferred_element_type=jnp.float32,
                        precision=lax.Precision.HIGHEST) * m
    c = lax.dot_general(b0_ref[...], out1t, (((0,), (1,)), ((), ())),
                        preferred_element_type=jnp.float32,
                        precision=lax.Precision.HIGHEST)[None, :] * m
    acc = jnp.dot(x_ref[0], n, preferred_element_type=jnp.float32)
    out = scale * (acc + c)
    out_ref[0] = out


SCHUNK = 512


def _logits_call(x, xg, maskf, W0, b0, W1, b1):
    return pl.pallas_call(
        _logits_body,
        grid=(B, S // SCHUNK),
        in_specs=[
            pl.BlockSpec((1, SCHUNK, E), lambda b, sc: (b, sc, 0)),
            pl.BlockSpec((1, NSLOT, E), lambda b, sc: (b, 0, 0)),
            pl.BlockSpec((1, 1, NSLOT), lambda b, sc: (b, 0, 0)),
            pl.BlockSpec((H, E), lambda b, sc: (0, 0)),
            pl.BlockSpec((H,), lambda b, sc: (0,)),
            pl.BlockSpec((H, E), lambda b, sc: (0, 0)),
            pl.BlockSpec((H,), lambda b, sc: (0,)),
        ],
        out_specs=pl.BlockSpec((1, SCHUNK, NSLOT), lambda b, sc: (b, sc, 0)),
        out_shape=jax.ShapeDtypeStruct((B, S, NSLOT), jnp.float32),
    )(x, xg, maskf, jnp.transpose(W0), b0, jnp.transpose(W1), b1)[:, :, :ROUT]


def kernel(x, sequences, W0, b0, W1, b1):
    flat_idx, maskf = _winner_call(sequences)
    xg = _gather_sc(x.reshape(B * S, E), flat_idx.reshape(B * NSLOT))
    return _logits_call(x, xg.reshape(B, NSLOT, E), maskf, W0, b0, W1, b1)


# speedup vs baseline: 1.4512x; 1.4512x over previous
import functools

import jax
import jax.numpy as jnp
from jax import lax
from jax.experimental import pallas as pl
from jax.experimental.pallas import tpu as pltpu
from jax.experimental.pallas import tpu_sc as plsc

RING_ID_START = 4
RING_ID_END = 104
NSLOT = 128
B, S, E, H = 16, 2048, 256, 64
ROUT = RING_ID_END - RING_ID_START

_NC, _NS = 2, 16
_NW = _NC * _NS
_BPW = (B * NSLOT) // _NW


def _winner_body(seq_ref, flat_ref, mask_ref):
    for b in range(B):
        row = seq_ref[b, :]
        valid = (row >= RING_ID_START) & (row <= RING_ID_END - 1)
        slot = jnp.where(valid, row - RING_ID_START, -1)
        slotb = jnp.broadcast_to(slot[None, :], (NSLOT, S))
        jcol = lax.broadcasted_iota(jnp.int32, (NSLOT, S), 0)
        siota = lax.broadcasted_iota(jnp.int32, (NSLOT, S), 1)
        vals = jnp.where(slotb == jcol, siota, -1)
        winner = jnp.max(vals, axis=1)
        flat_ref[b, :] = jnp.maximum(winner, 0) + b * S
        mask_ref[b, 0, :] = (winner >= 0).astype(jnp.float32)


def _winner_call(sequences):
    return pl.pallas_call(
        _winner_body,
        out_shape=(
            jax.ShapeDtypeStruct((B, NSLOT), jnp.int32),
            jax.ShapeDtypeStruct((B, 1, NSLOT), jnp.float32),
        ),
    )(sequences)


def _gather_sc_body(table_hbm, idx_hbm, out_hbm, idx_v, rows_v, sem):
    wid = lax.axis_index("s") * _NC + lax.axis_index("c")
    base = wid * _BPW
    pltpu.sync_copy(idx_hbm.at[pl.ds(base, _BPW)], idx_v)
    pltpu.async_copy(table_hbm.at[idx_v], rows_v, sem).wait()
    pltpu.sync_copy(rows_v, out_hbm.at[pl.ds(base, _BPW)])


@functools.cache
def _gather_sc_kernel():
    return pl.kernel(
        _gather_sc_body,
        mesh=plsc.VectorSubcoreMesh(core_axis_name="c", subcore_axis_name="s"),
        out_type=jax.ShapeDtypeStruct((B * NSLOT, E), jnp.float32),
        scratch_types=[
            pltpu.VMEM((_BPW,), jnp.int32),
            pltpu.VMEM((_BPW, E), jnp.float32),
            pltpu.SemaphoreType.DMA,
        ],
    )


def _gather_sc(table, idx):
    return _gather_sc_kernel()(table, idx)


def _fold_body(xg_ref, m_ref, w0t_ref, b0_ref, w1t_ref, b1_ref,
               n_ref, c_ref):
    scale = H ** -0.5
    xg = xg_ref[0]
    m = m_ref[0]
    out1t = lax.dot_general(xg, w1t_ref[...], (((1,), (1,)), ((), ())),
                            preferred_element_type=jnp.float32,
                            precision=lax.Precision.HIGHEST) + b1_ref[...]
    n_ref[0] = lax.dot_general(w0t_ref[...], out1t, (((0,), (1,)), ((), ())),
                               preferred_element_type=jnp.float32,
                               precision=lax.Precision.HIGHEST) * (m * scale)
    c_ref[0] = lax.dot_general(b0_ref[...], out1t, (((0,), (1,)), ((), ())),
                               preferred_element_type=jnp.float32,
                               precision=lax.Precision.HIGHEST)[None, :] * (
                                   m * scale)


def _fold_call(xg, maskf, W0, b0, W1, b1):
    return pl.pallas_call(
        _fold_body,
        grid=(B,),
        in_specs=[
            pl.BlockSpec((1, NSLOT, E), lambda b: (b, 0, 0)),
            pl.BlockSpec((1, 1, NSLOT), lambda b: (b, 0, 0)),
            pl.BlockSpec((H, E), lambda b: (0, 0)),
            pl.BlockSpec((H,), lambda b: (0,)),
            pl.BlockSpec((H, E), lambda b: (0, 0)),
            pl.BlockSpec((H,), lambda b: (0,)),
        ],
        out_specs=(
            pl.BlockSpec((1, E, NSLOT), lambda b: (b, 0, 0)),
            pl.BlockSpec((1, 1, NSLOT), lambda b: (b, 0, 0)),
        ),
        out_shape=(
            jax.ShapeDtypeStruct((B, E, NSLOT), jnp.float32),
            jax.ShapeDtypeStruct((B, 1, NSLOT), jnp.float32),
        ),
    )(xg, maskf, jnp.transpose(W0), b0, jnp.transpose(W1), b1)


def _logits_body(x_ref, n_ref, c_ref, out_ref):
    acc = jnp.dot(x_ref[0], n_ref[0], preferred_element_type=jnp.float32)
    out_ref[0] = acc + c_ref[0]


SCHUNK = 2048


def _logits_call(x, n_all, c_all):
    return pl.pallas_call(
        _logits_body,
        grid=(B, S // SCHUNK),
        in_specs=[
            pl.BlockSpec((1, SCHUNK, E), lambda b, sc: (b, sc, 0)),
            pl.BlockSpec((1, E, NSLOT), lambda b, sc: (b, 0, 0)),
            pl.BlockSpec((1, 1, NSLOT), lambda b, sc: (b, 0, 0)),
        ],
        out_specs=pl.BlockSpec((1, SCHUNK, NSLOT), lambda b, sc: (b, sc, 0)),
        out_shape=jax.ShapeDtypeStruct((B, S, NSLOT), jnp.float32),
    )(x, n_all, c_all)[:, :, :ROUT]


def kernel(x, sequences, W0, b0, W1, b1):
    flat_idx, maskf = _winner_call(sequences)
    xg = _gather_sc(x.reshape(B * S, E), flat_idx.reshape(B * NSLOT))
    n_all, c_all = _fold_call(xg.reshape(B, NSLOT, E), maskf, W0, b0, W1, b1)
    return _logits_call(x, n_all, c_all)


# speedup vs baseline: 1.5035x; 1.0360x over previous
import functools

import jax
import jax.numpy as jnp
from jax import lax
from jax.experimental import pallas as pl
from jax.experimental.pallas import tpu as pltpu
from jax.experimental.pallas import tpu_sc as plsc

RING_ID_START = 4
RING_ID_END = 104
NSLOT = 128
B, S, E, H = 16, 2048, 256, 64
ROUT = RING_ID_END - RING_ID_START

_NC, _NS = 2, 16
_NW = _NC * _NS
_BPW = (B * NSLOT) // _NW


def _winner_body(seq_ref, flat_ref, mask_ref):
    for b in range(B):
        row = seq_ref[b, :]
        valid = (row >= RING_ID_START) & (row <= RING_ID_END - 1)
        slot = jnp.where(valid, row - RING_ID_START, -1)
        slotb = jnp.broadcast_to(slot[None, :], (NSLOT, S))
        jcol = lax.broadcasted_iota(jnp.int32, (NSLOT, S), 0)
        siota = lax.broadcasted_iota(jnp.int32, (NSLOT, S), 1)
        vals = jnp.where(slotb == jcol, siota, -1)
        winner = jnp.max(vals, axis=1)
        flat_ref[b, :] = jnp.maximum(winner, 0) + b * S
        mask_ref[b, 0, :] = (winner >= 0).astype(jnp.float32)


def _winner_call(sequences):
    return pl.pallas_call(
        _winner_body,
        out_shape=(
            jax.ShapeDtypeStruct((B, NSLOT), jnp.int32),
            jax.ShapeDtypeStruct((B, 1, NSLOT), jnp.float32),
        ),
    )(sequences)


def _gather_sc_body(table_hbm, idx_hbm, out_hbm, idx_v, rows_v, sem):
    wid = lax.axis_index("s") * _NC + lax.axis_index("c")
    base = wid * _BPW
    pltpu.sync_copy(idx_hbm.at[pl.ds(base, _BPW)], idx_v)
    pltpu.async_copy(table_hbm.at[idx_v], rows_v, sem).wait()
    pltpu.sync_copy(rows_v, out_hbm.at[pl.ds(base, _BPW)])


@functools.cache
def _gather_sc_kernel():
    return pl.kernel(
        _gather_sc_body,
        mesh=plsc.VectorSubcoreMesh(core_axis_name="c", subcore_axis_name="s"),
        out_type=jax.ShapeDtypeStruct((B * NSLOT, E), jnp.float32),
        scratch_types=[
            pltpu.VMEM((_BPW,), jnp.int32),
            pltpu.VMEM((_BPW, E), jnp.float32),
            pltpu.SemaphoreType.DMA,
        ],
    )


def _gather_sc(table, idx):
    return _gather_sc_kernel()(table, idx)


def _fold_body(xg_ref, m_ref, w0t_ref, b0_ref, w1t_ref, b1_ref,
               n_ref, c_ref):
    scale = H ** -0.5
    for b in range(B):
        xg = xg_ref[b]
        m = m_ref[b]
        out1t = lax.dot_general(xg, w1t_ref[...], (((1,), (1,)), ((), ())),
                                preferred_element_type=jnp.float32,
                                precision=lax.Precision.HIGHEST) + b1_ref[...]
        n_ref[b] = lax.dot_general(w0t_ref[...], out1t,
                                   (((0,), (1,)), ((), ())),
                                   preferred_element_type=jnp.float32,
                                   precision=lax.Precision.HIGHEST) * (
                                       m * scale)
        c_ref[b] = lax.dot_general(b0_ref[...], out1t,
                                   (((0,), (1,)), ((), ())),
                                   preferred_element_type=jnp.float32,
                                   precision=lax.Precision.HIGHEST)[None, :] * (
                                       m * scale)


def _fold_call(xg, maskf, W0, b0, W1, b1):
    return pl.pallas_call(
        _fold_body,
        out_shape=(
            jax.ShapeDtypeStruct((B, E, NSLOT), jnp.float32),
            jax.ShapeDtypeStruct((B, 1, NSLOT), jnp.float32),
        ),
    )(xg, maskf, jnp.transpose(W0), b0, jnp.transpose(W1), b1)


def _logits_body(x_ref, n_ref, c_ref, out_ref):
    acc = jnp.dot(x_ref[0], n_ref[0], preferred_element_type=jnp.float32)
    out_ref[0] = acc + c_ref[0]


SCHUNK = 2048


def _logits_call(x, n_all, c_all):
    return pl.pallas_call(
        _logits_body,
        grid=(B, S // SCHUNK),
        in_specs=[
            pl.BlockSpec((1, SCHUNK, E), lambda b, sc: (b, sc, 0)),
            pl.BlockSpec((1, E, NSLOT), lambda b, sc: (b, 0, 0)),
            pl.BlockSpec((1, 1, NSLOT), lambda b, sc: (b, 0, 0)),
        ],
        out_specs=pl.BlockSpec((1, SCHUNK, NSLOT), lambda b, sc: (b, sc, 0)),
        out_shape=jax.ShapeDtypeStruct((B, S, NSLOT), jnp.float32),
    )(x, n_all, c_all)[:, :, :ROUT]


def kernel(x, sequences, W0, b0, W1, b1):
    flat_idx, maskf = _winner_call(sequences)
    xg = _gather_sc(x.reshape(B * S, E), flat_idx.reshape(B * NSLOT))
    n_all, c_all = _fold_call(xg.reshape(B, NSLOT, E), maskf, W0, b0, W1, b1)
    return _logits_call(x, n_all, c_all)


# speedup vs baseline: 1.9858x; 1.3208x over previous
import functools

import jax
import jax.numpy as jnp
from jax import lax
from jax.experimental import pallas as pl
from jax.experimental.pallas import tpu as pltpu
from jax.experimental.pallas import tpu_sc as plsc

RING_ID_START = 4
RING_ID_END = 104
NSLOT = 128
B, S, E, H = 16, 2048, 256, 64
ROUT = RING_ID_END - RING_ID_START

_NC, _NS = 2, 16
_NW = _NC * _NS
_BPW = (B * NSLOT) // _NW


def _winner_body(seq_ref, flat_ref, mask_ref):
    for b in range(B):
        row = seq_ref[b, :]
        valid = (row >= RING_ID_START) & (row <= RING_ID_END - 1)
        slot = jnp.where(valid, row - RING_ID_START, -1)
        slotb = jnp.broadcast_to(slot[None, :], (NSLOT, S))
        jcol = lax.broadcasted_iota(jnp.int32, (NSLOT, S), 0)
        siota = lax.broadcasted_iota(jnp.int32, (NSLOT, S), 1)
        vals = jnp.where(slotb == jcol, siota, -1)
        winner = jnp.max(vals, axis=1)
        flat_ref[b, :] = jnp.maximum(winner, 0) + b * S
        mask_ref[b, 0, :] = (winner >= 0).astype(jnp.float32)


def _winner_call(sequences):
    return pl.pallas_call(
        _winner_body,
        out_shape=(
            jax.ShapeDtypeStruct((B, NSLOT), jnp.int32),
            jax.ShapeDtypeStruct((B, 1, NSLOT), jnp.float32),
        ),
    )(sequences)


def _gather_sc_body(table_hbm, idx_hbm, out_hbm, idx_v, rows_v, sem):
    wid = lax.axis_index("s") * _NC + lax.axis_index("c")
    base = wid * _BPW
    pltpu.sync_copy(idx_hbm.at[pl.ds(base, _BPW)], idx_v)
    pltpu.async_copy(table_hbm.at[idx_v], rows_v, sem).wait()
    pltpu.sync_copy(rows_v, out_hbm.at[pl.ds(base, _BPW)])


@functools.cache
def _gather_sc_kernel():
    return pl.kernel(
        _gather_sc_body,
        mesh=plsc.VectorSubcoreMesh(core_axis_name="c", subcore_axis_name="s"),
        out_type=jax.ShapeDtypeStruct((B * NSLOT, E), jnp.float32),
        scratch_types=[
            pltpu.VMEM((_BPW,), jnp.int32),
            pltpu.VMEM((_BPW, E), jnp.float32),
            pltpu.SemaphoreType.DMA,
        ],
    )


def _gather_sc(table, idx):
    return _gather_sc_kernel()(table, idx)


def _fold_body(xg_ref, m_ref, w0t_ref, b0_ref, w1t_ref, b1_ref,
               n_ref, c_ref):
    scale = H ** -0.5
    for b in range(B):
        xg = xg_ref[b]
        m = m_ref[b]
        out1t = lax.dot_general(xg, w1t_ref[...], (((1,), (1,)), ((), ())),
                                preferred_element_type=jnp.float32,
                                precision=lax.Precision.HIGHEST) + b1_ref[...]
        n_ref[b] = lax.dot_general(w0t_ref[...], out1t,
                                   (((0,), (1,)), ((), ())),
                                   preferred_element_type=jnp.float32,
                                   precision=lax.Precision.HIGHEST) * (
                                       m * scale)
        crow = lax.dot_general(b0_ref[...], out1t,
                               (((0,), (1,)), ((), ())),
                               preferred_element_type=jnp.float32,
                               precision=lax.Precision.HIGHEST)[None, :] * (
                                   m * scale)
        c_ref[b] = jnp.transpose(crow, (1, 0))


def _fold_call(xg, maskf, W0, b0, W1, b1):
    return pl.pallas_call(
        _fold_body,
        out_shape=(
            jax.ShapeDtypeStruct((B, E, NSLOT), jnp.float32),
            jax.ShapeDtypeStruct((B, NSLOT, 1), jnp.float32),
        ),
    )(xg, maskf, jnp.transpose(W0), b0, jnp.transpose(W1), b1)


BG = 8
SCHUNK = 1024


def _logits_body(x_ref, n_ref, c_ref, out_ref):
    accs = []
    for bb in range(BG):
        acc_t = lax.dot_general(n_ref[bb], x_ref[bb],
                                (((0,), (1,)), ((), ())),
                                preferred_element_type=jnp.float32)
        accs.append(acc_t + c_ref[bb])
    stacked = jnp.stack(accs, axis=0)
    out_ref[...] = jnp.transpose(stacked, (1, 0, 2))[:ROUT]


def _logits_call(x, n_all, c_all):
    out = pl.pallas_call(
        _logits_body,
        grid=(B // BG, S // SCHUNK),
        in_specs=[
            pl.BlockSpec((BG, SCHUNK, E), lambda g, sc: (g, sc, 0)),
            pl.BlockSpec((BG, E, NSLOT), lambda g, sc: (g, 0, 0)),
            pl.BlockSpec((BG, NSLOT, 1), lambda g, sc: (g, 0, 0)),
        ],
        out_specs=pl.BlockSpec((ROUT, BG, SCHUNK), lambda g, sc: (0, g, sc)),
        out_shape=jax.ShapeDtypeStruct((ROUT, B, S), jnp.float32),
    )(x, n_all, c_all)
    return jnp.transpose(out, (1, 2, 0))


def kernel(x, sequences, W0, b0, W1, b1):
    flat_idx, maskf = _winner_call(sequences)
    xg = _gather_sc(x.reshape(B * S, E), flat_idx.reshape(B * NSLOT))
    n_all, c_all = _fold_call(xg.reshape(B, NSLOT, E), maskf, W0, b0, W1, b1)
    return _logits_call(x, n_all, c_all)
